# baseline (speedup 1.0000x reference)
import functools

import jax
import jax.numpy as jnp
from jax import lax
from jax.experimental import pallas as pl
from jax.experimental.pallas import tpu as pltpu
from jax.experimental.pallas import tpu_sc as plsc

_NC = 2
_NS = 16
_NW = _NC * _NS

_CP = 128
_VP = 1024
_CHUNK = 128


def _sc_gather_body(nrows, stage, table_hbm, idx_hbm, out_hbm, idx_v, rows_v, sem):
    wid = lax.axis_index("s") * _NC + lax.axis_index("c")
    base = wid * nrows
    pltpu.sync_copy(idx_hbm.at[pl.ds(base, nrows)], idx_v)
    sizes = []
    left = stage
    while left > 0:
        sizes.append(min(_CHUNK, left))
        left -= sizes[-1]
    for o in range(nrows // stage):
        descs = []
        r0 = 0
        for sz in sizes:
            descs.append(
                pltpu.async_copy(
                    table_hbm.at[idx_v.at[pl.ds(o * stage + r0, sz)]],
                    rows_v.at[pl.ds(r0, sz)],
                    sem,
                )
            )
            r0 += sz
        for desc in descs:
            desc.wait()
        pltpu.sync_copy(rows_v, out_hbm.at[pl.ds(base + o * stage, stage)])


def _make_sc_gather(n_rows_total):
    nrows = n_rows_total // _NW
    stage = min(nrows, 800)
    assert nrows % stage == 0
    mesh = plsc.VectorSubcoreMesh(core_axis_name="c", subcore_axis_name="s")
    return pl.kernel(
        functools.partial(_sc_gather_body, nrows, stage),
        mesh=mesh,
        out_type=jax.ShapeDtypeStruct((n_rows_total, _CP), jnp.float32),
        scratch_types=[
            pltpu.VMEM((nrows,), jnp.int32),
            pltpu.VMEM((stage, _CP), jnp.float32),
            pltpu.SemaphoreType.DMA,
        ],
    )


def _head_body(bb, t, h, x_ref, idx_ref, tab_ref, pos_ref, w_ref, b_ref, o_ref):
    i = pl.program_id(0)
    w = w_ref[...]
    bias = b_ref[...]
    pos = pos_ref[...]
    for j in range(bb):

        @pl.when(i < h)
        def _():
            ids = idx_ref[pl.ds(j * t, t), :]
            oh = (ids == lax.broadcasted_iota(jnp.int32, (t, _VP), 1)).astype(
                jnp.float32
            )
            tok = jnp.dot(oh, tab_ref[...], preferred_element_type=jnp.float32)
            o_ref[j] = (
                jnp.dot(tok + pos, w, preferred_element_type=jnp.float32) + bias
            )

        @pl.when(i >= h)
        def _():
            x = x_ref[pl.ds(j * t, t), :] + pos
            o_ref[j] = jnp.dot(x, w, preferred_element_type=jnp.float32) + bias


def kernel(idx, token_table, pos_table, W, b):
    B, T = idx.shape
    V, C = token_table.shape
    R = B * T
    idx_flat = idx.reshape(R).astype(jnp.int32)

    BB = 64
    grid = B // BB
    H = grid // 2

    table_p = jnp.pad(token_table, ((0, 0), (0, _CP - C)))
    tok_half = _make_sc_gather(R // 2)(table_p, idx_flat[R // 2 :])

    table_v = jnp.pad(table_p, ((0, _VP - V), (0, 0)))
    idx2 = idx_flat.reshape(R, 1)
    pos_p = jnp.pad(pos_table, ((0, 0), (0, _CP - C)))
    W_p = jnp.pad(W, ((0, _CP - C), (0, 0)))
    b2 = b.reshape(1, V)

    out = pl.pallas_call(
        functools.partial(_head_body, BB, T, H),
        grid=(grid,),
        in_specs=[
            pl.BlockSpec((BB * T, _CP), lambda i: (jnp.maximum(i - H, 0), 0)),
            pl.BlockSpec((BB * T, 1), lambda i: (i, 0)),
            pl.BlockSpec((_VP, _CP), lambda i: (0, 0)),
            pl.BlockSpec((T, _CP), lambda i: (0, 0)),
            pl.BlockSpec((_CP, V), lambda i: (0, 0)),
            pl.BlockSpec((1, V), lambda i: (0, 0)),
        ],
        out_specs=pl.BlockSpec((BB, T, V), lambda i: (i, 0, 0)),
        out_shape=jax.ShapeDtypeStruct((B, T, V), jnp.float32),
    )(tok_half, idx2, table_v, pos_p, W_p, b2)

    return out


# speedup vs baseline: 1.5833x; 1.5833x over previous
import functools

import jax
import jax.numpy as jnp
from jax import lax
from jax.experimental import pallas as pl
from jax.experimental.pallas import tpu as pltpu
from jax.experimental.pallas import tpu_sc as plsc

_NC = 2
_NS = 16
_NW = _NC * _NS

_CP = 128
_CHUNK = 128


def _sc_gather_body(nrows, stage, table_hbm, idx_hbm, out_hbm, idx_v, rows_v, sem):
    wid = lax.axis_index("s") * _NC + lax.axis_index("c")
    base = wid * nrows
    pltpu.sync_copy(idx_hbm.at[pl.ds(base, nrows)], idx_v)
    sizes = []
    left = stage
    while left > 0:
        sizes.append(min(_CHUNK, left))
        left -= sizes[-1]
    for o in range(nrows // stage):
        descs = []
        r0 = 0
        for sz in sizes:
            descs.append(
                pltpu.async_copy(
                    table_hbm.at[idx_v.at[pl.ds(o * stage + r0, sz)]],
                    rows_v.at[pl.ds(r0, sz)],
                    sem,
                )
            )
            r0 += sz
        for desc in descs:
            desc.wait()
        pltpu.sync_copy(rows_v, out_hbm.at[pl.ds(base + o * stage, stage)])


def _make_sc_gather(n_rows_total):
    nrows = n_rows_total // _NW
    stage = 800
    assert nrows % stage == 0
    mesh = plsc.VectorSubcoreMesh(core_axis_name="c", subcore_axis_name="s")
    return pl.kernel(
        functools.partial(_sc_gather_body, nrows, stage),
        mesh=mesh,
        out_type=jax.ShapeDtypeStruct((n_rows_total, _CP), jnp.float32),
        scratch_types=[
            pltpu.VMEM((nrows,), jnp.int32),
            pltpu.VMEM((stage, _CP), jnp.float32),
            pltpu.SemaphoreType.DMA,
        ],
    )


def _head_body(bb, t, x_ref, pos_ref, w_ref, b_ref, o_ref):
    w = w_ref[...]
    bias = b_ref[...]
    pos = pos_ref[...]
    for j in range(bb):
        x = x_ref[pl.ds(j * t, t), :] + pos
        o_ref[j] = jnp.dot(x, w, preferred_element_type=jnp.float32) + bias


def kernel(idx, token_table, pos_table, W, b):
    B, T = idx.shape
    V, C = token_table.shape
    R = B * T
    idx_flat = idx.reshape(R).astype(jnp.int32)

    table_p = jnp.pad(token_table, ((0, 0), (0, _CP - C)))
    tok = _make_sc_gather(R)(table_p, idx_flat)

    BB = 64
    pos_p = jnp.pad(pos_table, ((0, 0), (0, _CP - C)))
    W_p = jnp.pad(W, ((0, _CP - C), (0, 0)))
    b2 = b.reshape(1, V)

    grid = B // BB
    out = pl.pallas_call(
        functools.partial(_head_body, BB, T),
        grid=(grid,),
        in_specs=[
            pl.BlockSpec((BB * T, _CP), lambda i: (i, 0)),
            pl.BlockSpec((T, _CP), lambda i: (0, 0)),
            pl.BlockSpec((_CP, V), lambda i: (0, 0)),
            pl.BlockSpec((1, V), lambda i: (0, 0)),
        ],
        out_specs=pl.BlockSpec((BB, T, V), lambda i: (i, 0, 0)),
        out_shape=jax.ShapeDtypeStruct((B, T, V), jnp.float32),
    )(tok, pos_p, W_p, b2)

    return out
